# baseline (speedup 1.0000x reference)
import functools

import jax
import jax.numpy as jnp
from jax import lax
from jax.experimental import pallas as pl
from jax.experimental.pallas import tpu as pltpu
from jax.experimental.pallas import tpu_sc as plsc

NUM_ENTITIES = 1000000
NUM_RELATIONS = 1000
D = 64
B = 16384
NC, NS, L = 2, 16, 16
NW = NC * NS
BPW = B // NW


def _sqrt16(x):
    x = jnp.maximum(x, jnp.float32(1e-30))
    i = plsc.bitcast(x, jnp.int32)
    y = plsc.bitcast((i >> 1) + jnp.int32(0x1FBD1DF5), jnp.float32)
    half = jnp.float32(0.5)
    y = half * (y + x / y)
    y = half * (y + x / y)
    y = half * (y + x / y)
    return y


@functools.partial(
    pl.kernel,
    out_type=jax.ShapeDtypeStruct((B,), jnp.float32),
    mesh=plsc.VectorSubcoreMesh(core_axis_name="c", subcore_axis_name="s"),
    compiler_params=pltpu.CompilerParams(
        needs_layout_passes=False, use_tc_tiling_on_sc=False),
    scratch_types=[
        pltpu.VMEM((BPW,), jnp.int32),
        pltpu.VMEM((BPW,), jnp.int32),
        pltpu.VMEM((BPW,), jnp.int32),
        pltpu.VMEM((BPW, D), jnp.float32),
        pltpu.VMEM((BPW, D), jnp.float32),
        pltpu.VMEM((BPW, D), jnp.float32),
        pltpu.VMEM((BPW,), jnp.float32),
        pltpu.SemaphoreType.DMA,
        pltpu.SemaphoreType.DMA,
        pltpu.SemaphoreType.DMA,
    ],
)
def _transe_sc(ent_hbm, rel_hbm, heads_hbm, rels_hbm, tails_hbm, out_hbm,
               hidx, ridx, tidx, hrow, rrow, trow, scores,
               sem_h, sem_r, sem_t):
    wid = lax.axis_index("s") * NC + lax.axis_index("c")
    base = wid * BPW

    pltpu.sync_copy(heads_hbm.at[pl.ds(base, BPW)], hidx)
    pltpu.sync_copy(rels_hbm.at[pl.ds(base, BPW)], ridx)
    pltpu.sync_copy(tails_hbm.at[pl.ds(base, BPW)], tidx)

    ch = pltpu.async_copy(ent_hbm.at[hidx], hrow, sem_h)
    cr = pltpu.async_copy(rel_hbm.at[ridx], rrow, sem_r)
    ct = pltpu.async_copy(ent_hbm.at[tidx], trow, sem_t)
    ch.wait()
    cr.wait()
    ct.wait()

    lanes = lax.iota(jnp.int32, L)

    def group_body(g, carry):
        row0 = g * L
        rows = row0 + lanes

        def dim_body(d, acc):
            col = jnp.full((L,), d, jnp.int32)
            hv = plsc.load_gather(hrow, [rows, col])
            rv = plsc.load_gather(rrow, [rows, col])
            tv = plsc.load_gather(trow, [rows, col])
            dv = hv + rv - tv
            return acc + dv * dv

        acc = lax.fori_loop(0, D, dim_body, jnp.zeros((L,), jnp.float32))
        scores[pl.ds(row0, L)] = _sqrt16(acc)
        return carry

    lax.fori_loop(0, BPW // L, group_body, 0)
    pltpu.sync_copy(scores, out_hbm.at[pl.ds(base, BPW)])


def kernel(entity_emb, relation_emb, heads, relations, tails):
    return _transe_sc(
        entity_emb,
        relation_emb,
        heads.astype(jnp.int32),
        relations.astype(jnp.int32),
        tails.astype(jnp.int32),
    )


# speedup vs baseline: 1.0102x; 1.0102x over previous
import functools

import jax
import jax.numpy as jnp
from jax import lax
from jax.experimental import pallas as pl
from jax.experimental.pallas import tpu as pltpu
from jax.experimental.pallas import tpu_sc as plsc

NUM_ENTITIES = 1000000
NUM_RELATIONS = 1000
D = 64
B = 16384
NC, NS, L = 2, 16, 16
NW = NC * NS
BPW = B // NW
C = 128
NCHUNK = BPW // C


def _sqrt16(x):
    x = jnp.maximum(x, jnp.float32(1e-30))
    i = plsc.bitcast(x, jnp.int32)
    y = plsc.bitcast((i >> 1) + jnp.int32(0x1FBD1DF5), jnp.float32)
    half = jnp.float32(0.5)
    y = half * (y + x / y)
    y = half * (y + x / y)
    y = half * (y + x / y)
    return y


@functools.partial(
    pl.kernel,
    out_type=jax.ShapeDtypeStruct((B,), jnp.float32),
    mesh=plsc.VectorSubcoreMesh(core_axis_name="c", subcore_axis_name="s"),
    compiler_params=pltpu.CompilerParams(needs_layout_passes=False),
    scratch_types=[
        pltpu.VMEM((BPW,), jnp.int32),
        pltpu.VMEM((BPW,), jnp.int32),
        pltpu.VMEM((BPW,), jnp.int32),
        pltpu.VMEM((BPW,), jnp.int32),
        pltpu.VMEM((BPW,), jnp.int32),
        pltpu.VMEM((BPW,), jnp.int32),
        pltpu.VMEM((2, C, 2 * D), jnp.float32),
        pltpu.VMEM((2, C, 2 * D), jnp.float32),
        pltpu.VMEM((2, C, 2 * D), jnp.float32),
        pltpu.VMEM((BPW,), jnp.float32),
        pltpu.SemaphoreType.DMA,
        pltpu.SemaphoreType.DMA,
    ],
)
def _transe_sc(ent_hbm, rel_hbm, heads_hbm, rels_hbm, tails_hbm, out_hbm,
               hidx, ridx, tidx, hdiv, rdiv, tdiv, hbuf, rbuf, tbuf,
               scores, sem0, sem1):
    wid = lax.axis_index("s") * NC + lax.axis_index("c")
    base = wid * BPW

    pltpu.sync_copy(heads_hbm.at[pl.ds(base, BPW)], hidx)
    pltpu.sync_copy(rels_hbm.at[pl.ds(base, BPW)], ridx)
    pltpu.sync_copy(tails_hbm.at[pl.ds(base, BPW)], tidx)

    def shift_body(i, carry):
        off = pl.ds(i * L, L)
        hdiv[off] = hidx[off] >> 1
        rdiv[off] = ridx[off] >> 1
        tdiv[off] = tidx[off] >> 1
        return carry

    lax.fori_loop(0, BPW // L, shift_body, 0)

    sems = (sem0, sem1)

    def fire(c, slot):
        off = pl.ds(c * C, C)
        s = sems[slot]
        return (
            pltpu.async_copy(ent_hbm.at[hdiv.at[off]], hbuf.at[slot], s),
            pltpu.async_copy(rel_hbm.at[rdiv.at[off]], rbuf.at[slot], s),
            pltpu.async_copy(ent_hbm.at[tdiv.at[off]], tbuf.at[slot], s),
        )

    lanes = lax.iota(jnp.int32, L)

    def compute(c, slot):
        hb, rb, tb = hbuf.at[slot], rbuf.at[slot], tbuf.at[slot]

        def group_body(g, carry):
            rows = g * L + lanes
            r0 = c * C + g * L
            goff = pl.ds(r0, L)
            hcol = (hidx[goff] & 1) << 6
            rcol = (ridx[goff] & 1) << 6
            tcol = (tidx[goff] & 1) << 6
            acc = jnp.zeros((L,), jnp.float32)
            one = jnp.ones((L,), jnp.int32)
            for _ in range(D):
                hv = plsc.load_gather(hb, [rows, hcol])
                rv = plsc.load_gather(rb, [rows, rcol])
                tv = plsc.load_gather(tb, [rows, tcol])
                dv = hv + rv - tv
                acc = acc + dv * dv
                hcol = hcol + one
                rcol = rcol + one
                tcol = tcol + one
            scores[goff] = _sqrt16(acc)
            return carry

        lax.fori_loop(0, C // L, group_body, 0)

    descs = fire(0, 0)
    for c in range(NCHUNK):
        nxt = fire(c + 1, (c + 1) % 2) if c + 1 < NCHUNK else None
        for dsc in descs:
            dsc.wait()
        compute(c, c % 2)
        descs = nxt

    pltpu.sync_copy(scores, out_hbm.at[pl.ds(base, BPW)])


def kernel(entity_emb, relation_emb, heads, relations, tails):
    ent2 = entity_emb.reshape(NUM_ENTITIES // 2, 2 * D)
    rel2 = relation_emb.reshape(NUM_RELATIONS // 2, 2 * D)
    return _transe_sc(
        ent2,
        rel2,
        heads.astype(jnp.int32),
        relations.astype(jnp.int32),
        tails.astype(jnp.int32),
    )


# speedup vs baseline: 1.4008x; 1.3867x over previous
import functools

import jax
import jax.numpy as jnp
from jax import lax
from jax.experimental import pallas as pl
from jax.experimental.pallas import tpu as pltpu
from jax.experimental.pallas import tpu_sc as plsc

NUM_ENTITIES = 1000000
NUM_RELATIONS = 1000
D = 64
B = 16384
NC, NS, L = 2, 16, 16
NW = NC * NS
BPW = B // NW

NWIN = (NUM_ENTITIES + 127) // 128
WPW = (NWIN + NW - 1) // NW
EPW = WPW * 128
NREQ = 2 * B
NCHUNK = NREQ // L
CAP = 6144


def _sqrt16(x):
    x = jnp.maximum(x, jnp.float32(1e-30))
    i = plsc.bitcast(x, jnp.int32)
    y = plsc.bitcast((i >> 1) + jnp.int32(0x1FBD1DF5), jnp.float32)
    half = jnp.float32(0.5)
    y = half * (y + x / y)
    y = half * (y + x / y)
    y = half * (y + x / y)
    return y


@functools.partial(
    pl.kernel,
    out_type=jax.ShapeDtypeStruct((2 * B * D,), jnp.float32),
    mesh=plsc.VectorSubcoreMesh(core_axis_name="c", subcore_axis_name="s"),
    compiler_params=pltpu.CompilerParams(needs_layout_passes=False),
    scratch_types=[
        pltpu.VMEM((NREQ,), jnp.int32),
        pltpu.VMEM((CAP,), jnp.int32),
        pltpu.VMEM((CAP + L,), jnp.int32),
        pltpu.VMEM((L,), jnp.int32),
        pltpu.VMEM((2, 8, 8, 128), jnp.float32),
        pltpu.VMEM((2 * L * D,), jnp.float32),
        pltpu.SemaphoreType.DMA,
        pltpu.SemaphoreType.DMA,
        pltpu.SemaphoreType.DMA,
    ],
)
def _serve(ent_hbm, heads_hbm, tails_hbm, rows_hbm,
           idx_all, req, bkt, wlist, winbuf, stage, sem0, sem1, sem_row):
    wid = lax.axis_index("s") * NC + lax.axis_index("c")
    win_lo = wid * WPW
    nwin = jnp.minimum(WPW, NWIN - win_lo)
    e_lo = win_lo * 128
    e_hi = e_lo + EPW

    pltpu.sync_copy(heads_hbm, idx_all.at[pl.ds(0, B)])
    pltpu.sync_copy(tails_hbm, idx_all.at[pl.ds(B, B)])

    lanes = lax.iota(jnp.int32, L)
    sems = (sem0, sem1)

    def window_descs(lwin, slot):
        gwin = win_lo + lwin
        off = pl.multiple_of(gwin * 128, 128)
        return [
            pltpu.make_async_copy(
                ent_hbm.at[pl.ds(tr * 8, 8), pl.ds(off, 128)],
                winbuf.at[slot].at[tr], sems[slot])
            for tr in range(8)
        ]

    def fire_window(lwin, slot):
        @pl.when(lwin < nwin)
        def _():
            for dsc in window_descs(lwin, slot):
                dsc.start()

    def wait_window(lwin, slot):
        @pl.when(lwin < nwin)
        def _():
            for dsc in window_descs(lwin, slot):
                dsc.wait()

    def drain_rows(n):
        for l in range(L):
            @pl.when(l < n)
            def _(l=l):
                pltpu.make_async_copy(
                    rows_hbm.at[pl.ds(0, D)],
                    stage.at[pl.ds(l * D, D)], sem_row).wait()

    dbase = []
    for q in range(4):
        dq = lanes + q * L
        dbase.append(((dq >> 3) << 10) | ((dq & 7) << 7))

    def extract_window(lwin, slot, bcnt, st):
        wbuf = winbuf.at[slot]

        def chunk_body(ch, st):
            gch, n1, n2 = st
            p = gch & 1
            drain_rows(n2)
            v = bkt[pl.ds(ch * L, L)]
            win16 = (v >> 16) >> 7
            m = (win16 == lwin) & (ch * L + lanes < bcnt)
            nsel = plsc.all_reduce_population_count(m)[0]
            plsc.store_compressed(wlist.at[pl.ds(0, L)], v, mask=m)
            wv = wlist[...]
            for l in range(L):
                @pl.when(l < nsel)
                def _(l=l):
                    item = wv[l]
                    e_in_win = (item >> 16) & 127
                    dst_b = item & 0x7FFF
                    soff = p * (L * D) + l * D
                    for q in range(4):
                        col = dbase[q] + e_in_win
                        vals = plsc.load_gather(wbuf, [col >> 10,
                                                       (col >> 7) & 7,
                                                       col & 127])
                        stage[pl.ds(soff + q * L, L)] = vals
                    pltpu.async_copy(
                        stage.at[pl.ds(soff, D)],
                        rows_hbm.at[pl.ds(dst_b * D, D)], sem_row)
            return gch + 1, nsel, n1

        nch = (bcnt + L - 1) // L
        return lax.fori_loop(0, nch, chunk_body, st)

    def serve_batch(cnt, st):
        def group_body(g, st):
            def bf_body(ch, bcnt):
                v = req[pl.ds(ch * L, L)]
                win16 = (v >> 16) >> 7
                m = ((win16 >> 4) == g) & (ch * L + lanes < cnt)
                plsc.store_compressed(bkt.at[pl.ds(bcnt, L)], v, mask=m)
                return bcnt + plsc.all_reduce_population_count(m)[0]

            nch = (cnt + L - 1) // L
            bcnt = lax.fori_loop(0, nch, bf_body, jnp.int32(0))

            def pair_body(i, st):
                l0 = g * 16 + 2 * i
                wait_window(l0, 0)
                st = extract_window(l0, 0, bcnt, st)
                fire_window(l0 + 2, 0)
                wait_window(l0 + 1, 1)
                st = extract_window(l0 + 1, 1, bcnt, st)
                fire_window(l0 + 3, 1)
                return st

            return lax.fori_loop(0, 8, pair_body, st)

        fire_window(0, 0)
        fire_window(1, 1)
        return lax.fori_loop(0, 16, group_body, st)

    def outer_cond(st):
        return st[0] < NCHUNK

    def outer_body(st):
        ch0, gch, n1, n2 = st

        def f_cond(st2):
            ch, cnt = st2
            return (ch < NCHUNK) & (cnt <= CAP - L)

        def f_body(st2):
            ch, cnt = st2
            e16 = idx_all[pl.ds(ch * L, L)]
            m = (e16 >= e_lo) & (e16 < e_hi)
            packed = ((e16 - e_lo) << 16) | (ch * L + lanes)
            plsc.store_compressed(req.at[pl.ds(cnt, L)], packed, mask=m)
            return ch + 1, cnt + plsc.all_reduce_population_count(m)[0]

        ch1, cnt = lax.while_loop(f_cond, f_body, (ch0, jnp.int32(0)))
        gch, n1, n2 = serve_batch(cnt, (gch, n1, n2))
        return ch1, gch, n1, n2

    st = lax.while_loop(
        outer_cond, outer_body,
        (jnp.int32(0), jnp.int32(0), jnp.int32(0), jnp.int32(0)))
    drain_rows(st[3])
    drain_rows(st[2])


@functools.partial(
    pl.kernel,
    out_type=jax.ShapeDtypeStruct((B,), jnp.float32),
    mesh=plsc.VectorSubcoreMesh(core_axis_name="c", subcore_axis_name="s"),
    compiler_params=pltpu.CompilerParams(needs_layout_passes=False),
    scratch_types=[
        pltpu.VMEM((BPW * D,), jnp.float32),
        pltpu.VMEM((BPW * D,), jnp.float32),
        pltpu.VMEM((BPW,), jnp.int32),
        pltpu.VMEM((BPW,), jnp.int32),
        pltpu.VMEM((BPW // 2, 2 * D), jnp.float32),
        pltpu.VMEM((BPW,), jnp.float32),
        pltpu.SemaphoreType.DMA,
        pltpu.SemaphoreType.DMA,
    ],
)
def _score(rows_hbm, rel_hbm, rels_hbm, out_hbm,
           hrow, trow, ridx, rdiv, rbuf, scores, sem_ht, sem_r):
    wid = lax.axis_index("s") * NC + lax.axis_index("c")
    base = wid * BPW
    C = BPW // 2

    ch = pltpu.async_copy(rows_hbm.at[pl.ds(base * D, BPW * D)],
                          hrow, sem_ht)
    ct = pltpu.async_copy(rows_hbm.at[pl.ds((B + base) * D, BPW * D)],
                          trow, sem_ht)
    pltpu.sync_copy(rels_hbm.at[pl.ds(base, BPW)], ridx)

    def shift_body(i, carry):
        off = pl.ds(i * L, L)
        rdiv[off] = ridx[off] >> 1
        return carry

    lax.fori_loop(0, BPW // L, shift_body, 0)
    ch.wait()
    ct.wait()

    lanes = lax.iota(jnp.int32, L)

    def compute_chunk(c):
        cr = pltpu.async_copy(rel_hbm.at[rdiv.at[pl.ds(c * C, C)]],
                              rbuf, sem_r)
        cr.wait()

        def group_body(g, carry):
            bl = c * C + g * L
            goff = pl.ds(bl, L)
            rows_l = g * L + lanes
            htbase = (bl + lanes) << 6
            rcol = (ridx[goff] & 1) << 6
            acc = jnp.zeros((L,), jnp.float32)
            one = jnp.ones((L,), jnp.int32)
            hti = htbase
            for _ in range(D):
                hv = plsc.load_gather(hrow, [hti])
                tv = plsc.load_gather(trow, [hti])
                rv = plsc.load_gather(rbuf, [rows_l, rcol])
                dv = hv + rv - tv
                acc = acc + dv * dv
                hti = hti + one
                rcol = rcol + one
            scores[goff] = _sqrt16(acc)
            return carry

        lax.fori_loop(0, C // L, group_body, 0)

    compute_chunk(0)
    compute_chunk(1)
    pltpu.sync_copy(scores, out_hbm.at[pl.ds(base, BPW)])


def kernel(entity_emb, relation_emb, heads, relations, tails):
    rows = _serve(
        entity_emb.T,
        heads.astype(jnp.int32),
        tails.astype(jnp.int32),
    )
    return _score(
        rows,
        relation_emb.reshape(NUM_RELATIONS // 2, 2 * D),
        relations.astype(jnp.int32),
    )
